# baseline (speedup 1.0000x reference)
import functools

import jax
import jax.numpy as jnp
from jax import lax
from jax.experimental import pallas as pl
from jax.experimental.pallas import tpu as pltpu
from jax.experimental.pallas import tpu_sc as plsc

N_NODES = 10000
N_EDGES = 320000
D = 128
L = 16
NS = 16
ES = N_EDGES // 2
RW = 80
CE = RW * 16
NCH = ES // CE
NROW = ES // RW
RPT = NROW // NS
WPN = D // 32


@functools.lru_cache(maxsize=1)
def _build():
    mesh = plsc.VectorSubcoreMesh(core_axis_name="c", subcore_axis_name="s")

    @functools.partial(
        pl.kernel,
        mesh=mesh,
        compiler_params=pltpu.CompilerParams(needs_layout_passes=False,
                                             use_tc_tiling_on_sc=False),
        out_type=jax.ShapeDtypeStruct((N_EDGES // RW, RW), jnp.float32),
        scratch_types=[
            pltpu.VMEM((N_NODES * WPN,), jnp.int32),
            pltpu.VMEM((CE,), jnp.int32), pltpu.VMEM((CE,), jnp.int32),
            pltpu.VMEM((CE,), jnp.int32), pltpu.VMEM((CE,), jnp.int32),
            pltpu.VMEM((16, RW), jnp.float32),
            pltpu.VMEM((16,), jnp.int32),
            pltpu.VMEM((RPT, RW), jnp.float32),
            pltpu.VMEM_SHARED((NROW, RW), jnp.float32),
            pltpu.SemaphoreType.DMA, pltpu.SemaphoreType.DMA,
        ],
    )
    def sc_kernel(zt_hbm, src_hbm, dst_hbm, out_hbm,
                  zslice_v, sidx0, sidx1, didx0, didx1,
                  part_v, rowidx_v, zero_v, accum, isem0, isem1):
        cid = lax.axis_index("c")
        sid = lax.axis_index("s")
        lane = lax.iota(jnp.int32, 16)

        sidx = (sidx0, sidx1)
        didx = (didx0, didx1)
        isem = (isem0, isem1)

        pltpu.sync_copy(zt_hbm.at[sid], zslice_v)

        zvec = jnp.zeros((16,), jnp.float32)

        def zrow(r, carry):
            for k in range(RW // L):
                zero_v[r, pl.ds(k * L, L)] = zvec
            return carry

        lax.fori_loop(0, RPT, zrow, 0)
        pltpu.sync_copy(zero_v, accum.at[pl.ds(sid * RPT, RPT)])

        ebase = cid * ES

        def issue(c, b):
            off = ebase + c * CE
            pltpu.async_copy(src_hbm.at[pl.ds(off, CE)], sidx[b], isem[b])
            pltpu.async_copy(dst_hbm.at[pl.ds(off, CE)], didx[b], isem[b])

        def wait(b):
            pltpu.make_async_copy(src_hbm.at[pl.ds(0, CE)], sidx[b], isem[b]).wait()
            pltpu.make_async_copy(dst_hbm.at[pl.ds(0, CE)], didx[b], isem[b]).wait()

        plsc.subcore_barrier()

        def compute(c, b):
            sic, dic = sidx[b], didx[b]

            @plsc.parallel_loop(0, CE // L, 1, unroll=4)
            def group_body(g):
                s16 = sic[pl.ds(g * L, L)] * WPN
                d16 = dic[pl.ds(g * L, L)] * WPN
                acc = jnp.zeros((16,), jnp.float32)
                for w in range(WPN):
                    sw = plsc.load_gather(zslice_v, [s16 + w])
                    dw = plsc.load_gather(zslice_v, [d16 + w])
                    p = plsc.bitcast(sw, jnp.bfloat16) * plsc.bitcast(dw, jnp.bfloat16)
                    pa, pb = plsc.unpack(p, format=plsc.PackFormat.INTERLEAVED)
                    acc = acc + (pa + pb)
                part_v[g // (RW // L), pl.ds((g % (RW // L)) * L, L)] = acc

            rowidx_v[...] = lane + c * 16
            pltpu.sync_copy(part_v, accum.at[rowidx_v], add=True)

        issue(0, 0)

        def pair_body(i, carry):
            c = 2 * i
            issue(c + 1, 1)
            wait(0)
            compute(c, 0)
            issue(c + 2, 0)
            wait(1)
            compute(c + 1, 1)
            return carry

        lax.fori_loop(0, (NCH - 1) // 2, pair_body, 0)
        wait(0)
        compute(NCH - 1, 0)

        plsc.subcore_barrier()
        pltpu.sync_copy(accum.at[pl.ds(sid * RPT, RPT)],
                        out_hbm.at[pl.ds(cid * NROW + sid * RPT, RPT)])

    return sc_kernel


def kernel(z, edge_index):
    ei = edge_index.astype(jnp.int32)
    zb = z.astype(jnp.bfloat16)
    zt = zb.reshape(N_NODES, NS, 8).transpose(1, 0, 2)
    zi = jax.lax.bitcast_convert_type(zt.reshape(NS, N_NODES, 4, 2), jnp.int32)
    zi = zi.reshape(NS, N_NODES * WPN)
    out2d = _build()(zi, ei[0], ei[1])
    return out2d.reshape(N_EDGES)


# speedup vs baseline: 1.0085x; 1.0085x over previous
import functools

import jax
import jax.numpy as jnp
from jax import lax
from jax.experimental import pallas as pl
from jax.experimental.pallas import tpu as pltpu
from jax.experimental.pallas import tpu_sc as plsc

N_NODES = 10000
N_EDGES = 320000
D = 128
L = 16
NS = 16
ES = N_EDGES // 2
RW = 80
CE = RW * 16
NCH = ES // CE
NROW = ES // RW
RPT = NROW // NS
WPN = D // 32


@functools.lru_cache(maxsize=1)
def _build():
    mesh = plsc.VectorSubcoreMesh(core_axis_name="c", subcore_axis_name="s")

    @functools.partial(
        pl.kernel,
        mesh=mesh,
        compiler_params=pltpu.CompilerParams(needs_layout_passes=False,
                                             use_tc_tiling_on_sc=False),
        out_type=jax.ShapeDtypeStruct((N_EDGES // RW, RW), jnp.float32),
        scratch_types=[
            pltpu.VMEM((N_NODES * WPN,), jnp.int32),
            pltpu.VMEM((CE,), jnp.int32), pltpu.VMEM((CE,), jnp.int32),
            pltpu.VMEM((CE,), jnp.int32), pltpu.VMEM((CE,), jnp.int32),
            pltpu.VMEM((16, RW), jnp.float32),
            pltpu.VMEM((16,), jnp.int32),
            pltpu.VMEM((RPT, RW), jnp.float32),
            pltpu.VMEM_SHARED((NROW, RW), jnp.float32),
            pltpu.SemaphoreType.DMA, pltpu.SemaphoreType.DMA,
        ],
    )
    def sc_kernel(zt_hbm, src_hbm, dst_hbm, out_hbm,
                  zslice_v, sidx0, sidx1, didx0, didx1,
                  part_v, rowidx_v, zero_v, accum, isem0, isem1):
        cid = lax.axis_index("c")
        sid = lax.axis_index("s")
        lane = lax.iota(jnp.int32, 16)

        sidx = (sidx0, sidx1)
        didx = (didx0, didx1)
        isem = (isem0, isem1)

        pltpu.sync_copy(zt_hbm.at[sid], zslice_v)

        zvec = jnp.zeros((16,), jnp.float32)

        def zrow(r, carry):
            for k in range(RW // L):
                zero_v[r, pl.ds(k * L, L)] = zvec
            return carry

        lax.fori_loop(0, RPT, zrow, 0)
        pltpu.sync_copy(zero_v, accum.at[pl.ds(sid * RPT, RPT)])

        ebase = cid * ES

        def cc(c):
            return lax.rem(c + sid * 8, NCH)

        def issue(c, b):
            off = ebase + cc(c) * CE
            pltpu.async_copy(src_hbm.at[pl.ds(off, CE)], sidx[b], isem[b])
            pltpu.async_copy(dst_hbm.at[pl.ds(off, CE)], didx[b], isem[b])

        def wait(b):
            pltpu.make_async_copy(src_hbm.at[pl.ds(0, CE)], sidx[b], isem[b]).wait()
            pltpu.make_async_copy(dst_hbm.at[pl.ds(0, CE)], didx[b], isem[b]).wait()

        plsc.subcore_barrier()

        def compute(c, b):
            sic, dic = sidx[b], didx[b]

            @plsc.parallel_loop(0, CE // L, 1, unroll=4)
            def group_body(g):
                s16 = sic[pl.ds(g * L, L)] * WPN
                d16 = dic[pl.ds(g * L, L)] * WPN
                acc = jnp.zeros((16,), jnp.float32)
                for w in range(WPN):
                    sw = plsc.load_gather(zslice_v, [s16 + w])
                    dw = plsc.load_gather(zslice_v, [d16 + w])
                    p = plsc.bitcast(sw, jnp.bfloat16) * plsc.bitcast(dw, jnp.bfloat16)
                    pa, pb = plsc.unpack(p, format=plsc.PackFormat.INTERLEAVED)
                    acc = acc + (pa + pb)
                part_v[g // (RW // L), pl.ds((g % (RW // L)) * L, L)] = acc

            rowidx_v[...] = lane + cc(c) * 16
            pltpu.sync_copy(part_v, accum.at[rowidx_v], add=True)

        issue(0, 0)

        def pair_body(i, carry):
            c = 2 * i
            issue(c + 1, 1)
            wait(0)
            compute(c, 0)
            issue(c + 2, 0)
            wait(1)
            compute(c + 1, 1)
            return carry

        lax.fori_loop(0, (NCH - 1) // 2, pair_body, 0)
        wait(0)
        compute(NCH - 1, 0)

        plsc.subcore_barrier()
        pltpu.sync_copy(accum.at[pl.ds(sid * RPT, RPT)],
                        out_hbm.at[pl.ds(cid * NROW + sid * RPT, RPT)])

    return sc_kernel


def kernel(z, edge_index):
    ei = edge_index.astype(jnp.int32)
    zb = z.astype(jnp.bfloat16)
    zt = zb.reshape(N_NODES, NS, 8).transpose(1, 0, 2)
    zi = jax.lax.bitcast_convert_type(zt.reshape(NS, N_NODES, 4, 2), jnp.int32)
    zi = zi.reshape(NS, N_NODES * WPN)
    out2d = _build()(zi, ei[0], ei[1])
    return out2d.reshape(N_EDGES)


# speedup vs baseline: 1.3607x; 1.3492x over previous
import functools

import jax
import jax.numpy as jnp
from jax import lax
from jax.experimental import pallas as pl
from jax.experimental.pallas import tpu as pltpu
from jax.experimental.pallas import tpu_sc as plsc

N_NODES = 10000
N_EDGES = 320000
D = 128
L = 16
NW = 32
E_W = N_EDGES // NW
CH = 80
NCHUNK = E_W // CH


@functools.lru_cache(maxsize=1)
def _build():
    mesh = plsc.VectorSubcoreMesh(core_axis_name="c", subcore_axis_name="s")

    @functools.partial(
        pl.kernel,
        mesh=mesh,
        compiler_params=pltpu.CompilerParams(needs_layout_passes=False,
                                             use_tc_tiling_on_sc=False),
        out_type=jax.ShapeDtypeStruct((N_EDGES,), jnp.float32),
        scratch_types=[
            pltpu.VMEM((E_W,), jnp.int32),
            pltpu.VMEM((E_W,), jnp.int32),
            pltpu.VMEM((CH, D // 2), jnp.int32), pltpu.VMEM((CH, D // 2), jnp.int32),
            pltpu.VMEM((CH, D // 2), jnp.int32), pltpu.VMEM((CH, D // 2), jnp.int32),
            pltpu.VMEM((E_W,), jnp.float32),
            pltpu.VMEM_SHARED((N_NODES, D // 2), jnp.int32),
            pltpu.SemaphoreType.DMA, pltpu.SemaphoreType.DMA,
        ],
    )
    def sc_kernel(z_hbm, src_hbm, dst_hbm, out_hbm,
                  sidx_v, didx_v,
                  srows0, srows1, drows0, drows1,
                  out_v, zs, gsem0, gsem1):
        wid = lax.axis_index("s") * 2 + lax.axis_index("c")
        base = wid * E_W
        lane = lax.iota(jnp.int32, 16)
        lane0 = lane == 0

        srows = (srows0, srows1)
        drows = (drows0, drows1)
        gsem = (gsem0, gsem1)

        @pl.when(lax.axis_index("s") == 0)
        def _stage():
            pltpu.sync_copy(z_hbm, zs)

        pltpu.sync_copy(src_hbm.at[pl.ds(base, E_W)], sidx_v)
        pltpu.sync_copy(dst_hbm.at[pl.ds(base, E_W)], didx_v)
        plsc.subcore_barrier()

        def issue(c, b):
            off = c * CH
            pltpu.async_copy(zs.at[sidx_v.at[pl.ds(off, CH)]], srows[b], gsem[b])
            pltpu.async_copy(zs.at[didx_v.at[pl.ds(off, CH)]], drows[b], gsem[b])

        def wait(b):
            pltpu.make_async_copy(z_hbm.at[pl.ds(0, CH)], srows[b], gsem[b]).wait()
            pltpu.make_async_copy(z_hbm.at[pl.ds(0, CH)], drows[b], gsem[b]).wait()

        def compute(c, b):
            sr, dr = srows[b], drows[b]
            ebase = c * CH

            @plsc.parallel_loop(0, CH, 1, unroll=8)
            def edge_body(e):
                part32 = None
                for k in range(D // 32):
                    sv = plsc.bitcast(sr[e, pl.ds(k * L, L)], jnp.bfloat16)
                    dv = plsc.bitcast(dr[e, pl.ds(k * L, L)], jnp.bfloat16)
                    p = sv * dv
                    part32 = p if part32 is None else part32 + p
                pa, pb = plsc.unpack(part32, format=plsc.PackFormat.INTERLEAVED)
                r = jnp.sum(pa + pb)
                plsc.store_scatter(out_v, [lane * 0 + (ebase + e)],
                                   jnp.zeros((16,), jnp.float32) + r,
                                   mask=lane0)

        issue(0, 0)

        def pair_body(i, carry):
            c = 2 * i
            issue(c + 1, 1)
            wait(0)
            compute(c, 0)
            issue(c + 2, 0)
            wait(1)
            compute(c + 1, 1)
            return carry

        lax.fori_loop(0, (NCHUNK - 1) // 2, pair_body, 0)
        wait(0)
        compute(NCHUNK - 1, 0)
        pltpu.sync_copy(out_v, out_hbm.at[pl.ds(base, E_W)])

    return sc_kernel


def kernel(z, edge_index):
    ei = edge_index.astype(jnp.int32)
    zb = z.astype(jnp.bfloat16)
    zi = jax.lax.bitcast_convert_type(zb.reshape(N_NODES, D // 2, 2), jnp.int32)
    return _build()(zi, ei[0], ei[1])
